# baseline (speedup 1.0000x reference)
import functools

import numpy as np
import jax
import jax.numpy as jnp
from jax import lax
from jax.experimental import pallas as pl
from jax.experimental.pallas import tpu as pltpu
from jax.experimental.pallas import tpu_sc as plsc

_BATCH = 16384
_NUM_FEATURE = 26
_EMB = 16
_BINS = 50
_TOTAL = 100
_INV = 3

_N = _BATCH * _NUM_FEATURE
_IPS = 128
_NROWS = _N // _IPS
_K = 8

_j = np.arange(_BINS)[:, None]
_c = np.arange(3 * _TOTAL)[None, :]
_MHB = (np.abs(_c - (_j + _TOTAL)) <= _INV).astype(np.float32)


def _table_body(mhb_ref, w_ref, e_ref):
    e_ref[...] = lax.dot_general(
        mhb_ref[...], w_ref[...], (((1,), (1,)), ((), ())),
        preferred_element_type=jnp.float32)


def kernel(x, W):
    table = pl.pallas_call(
        _table_body,
        out_shape=jax.ShapeDtypeStruct((_BINS, _EMB), jnp.float32),
    )(jnp.asarray(_MHB), W)

    idx = x.astype(jnp.int32).reshape(_N)

    info = plsc.get_sparse_core_info()
    nc, ns = info.num_cores, info.num_subcores
    nw = nc * ns
    per_w = _N // nw
    n_chunks = 4
    chunk = per_w // n_chunks

    mesh = plsc.VectorSubcoreMesh(core_axis_name="c", subcore_axis_name="s")

    @functools.partial(
        pl.kernel,
        out_type=jax.ShapeDtypeStruct((_N, _EMB), jnp.float32),
        mesh=mesh,
        scratch_types=[
            pltpu.VMEM((per_w,), jnp.int32),
            pltpu.VMEM((2, chunk, _EMB), jnp.float32),
            pltpu.SemaphoreType.DMA,
            pltpu.SemaphoreType.DMA,
        ],
        compiler_params=pltpu.CompilerParams(use_tc_tiling_on_sc=False),
    )
    def _gather(tab_hbm, idx_hbm, out_hbm, idx_v, rows_v, sem0, sem1):
        wid = lax.axis_index("s") * nc + lax.axis_index("c")
        base = wid * per_w
        pltpu.sync_copy(idx_hbm.at[pl.ds(base, per_w)], idx_v)

        sems = [sem0, sem1]
        handles = [None, None]
        handles[0] = pltpu.async_copy(
            tab_hbm.at[idx_v.at[pl.ds(0, chunk)]], rows_v.at[0], sems[0])
        for c in range(n_chunks):
            nb = (c + 1) % 2
            if c + 1 < n_chunks:
                handles[nb] = pltpu.async_copy(
                    tab_hbm.at[idx_v.at[pl.ds((c + 1) * chunk, chunk)]],
                    rows_v.at[nb], sems[nb])
            handles[c % 2].wait()
            pltpu.sync_copy(rows_v.at[c % 2],
                            out_hbm.at[pl.ds(base + c * chunk, chunk)])

    out = _gather(table, idx)
    return out.reshape(_BATCH, _NUM_FEATURE * _EMB)


# speedup vs baseline: 3.3919x; 3.3919x over previous
import functools

import numpy as np
import jax
import jax.numpy as jnp
from jax import lax
from jax.experimental import pallas as pl
from jax.experimental.pallas import tpu as pltpu
from jax.experimental.pallas import tpu_sc as plsc

_BATCH = 16384
_NUM_FEATURE = 26
_EMB = 16
_BINS = 50
_TOTAL = 100
_INV = 3

_N = _BATCH * _NUM_FEATURE
_IPS = 128
_NROWS = _N // _IPS
_K = 8

_j = np.arange(_BINS)[:, None]
_c = np.arange(3 * _TOTAL)[None, :]
_MHB = (np.abs(_c - (_j + _TOTAL)) <= _INV).astype(np.float32)


def _table_body(mhb_ref, w_ref, e_ref):
    e_ref[...] = lax.dot_general(
        mhb_ref[...], w_ref[...], (((1,), (1,)), ((), ())),
        preferred_element_type=jnp.float32)


def kernel(x, W):
    table = pl.pallas_call(
        _table_body,
        out_shape=jax.ShapeDtypeStruct((_BINS, _EMB), jnp.float32),
    )(jnp.asarray(_MHB), W)

    reps = 128
    table_rep = jnp.tile(table, (reps, 1))
    lane = (jnp.arange(_N, dtype=jnp.int32) % reps) * _BINS
    idx = x.astype(jnp.int32).reshape(_N) + lane

    info = plsc.get_sparse_core_info()
    nc, ns = info.num_cores, info.num_subcores
    nw = nc * ns
    per_w = _N // nw
    n_chunks = 4
    chunk = per_w // n_chunks

    mesh = plsc.VectorSubcoreMesh(core_axis_name="c", subcore_axis_name="s")

    @functools.partial(
        pl.kernel,
        out_type=jax.ShapeDtypeStruct((_N, _EMB), jnp.float32),
        mesh=mesh,
        scratch_types=[
            pltpu.VMEM((per_w,), jnp.int32),
            pltpu.VMEM((2, chunk, _EMB), jnp.float32),
            pltpu.SemaphoreType.DMA,
            pltpu.SemaphoreType.DMA,
        ],
        compiler_params=pltpu.CompilerParams(use_tc_tiling_on_sc=False),
    )
    def _gather(tab_hbm, idx_hbm, out_hbm, idx_v, rows_v, sem0, sem1):
        wid = lax.axis_index("s") * nc + lax.axis_index("c")
        base = wid * per_w
        pltpu.sync_copy(idx_hbm.at[pl.ds(base, per_w)], idx_v)

        sems = [sem0, sem1]
        handles = [None, None]
        handles[0] = pltpu.async_copy(
            tab_hbm.at[idx_v.at[pl.ds(0, chunk)]], rows_v.at[0], sems[0])
        for c in range(n_chunks):
            nb = (c + 1) % 2
            if c + 1 < n_chunks:
                handles[nb] = pltpu.async_copy(
                    tab_hbm.at[idx_v.at[pl.ds((c + 1) * chunk, chunk)]],
                    rows_v.at[nb], sems[nb])
            handles[c % 2].wait()
            pltpu.sync_copy(rows_v.at[c % 2],
                            out_hbm.at[pl.ds(base + c * chunk, chunk)])

    out = _gather(table_rep, idx)
    return out.reshape(_BATCH, _NUM_FEATURE * _EMB)


# speedup vs baseline: 3.5328x; 1.0416x over previous
import functools

import numpy as np
import jax
import jax.numpy as jnp
from jax import lax
from jax.experimental import pallas as pl
from jax.experimental.pallas import tpu as pltpu
from jax.experimental.pallas import tpu_sc as plsc

_BATCH = 16384
_NUM_FEATURE = 26
_EMB = 16
_BINS = 50
_TOTAL = 100
_INV = 3

_N = _BATCH * _NUM_FEATURE
_IPS = 128
_NROWS = _N // _IPS
_K = 8

_j = np.arange(_BINS)[:, None]
_c = np.arange(3 * _TOTAL)[None, :]
_MHB = (np.abs(_c - (_j + _TOTAL)) <= _INV).astype(np.float32)


def _table_body(mhb_ref, w_ref, e_ref):
    e_ref[...] = lax.dot_general(
        mhb_ref[...], w_ref[...], (((1,), (1,)), ((), ())),
        preferred_element_type=jnp.float32)


def kernel(x, W):
    table = pl.pallas_call(
        _table_body,
        out_shape=jax.ShapeDtypeStruct((_BINS, _EMB), jnp.float32),
    )(jnp.asarray(_MHB), W)

    reps = 128
    table_rep = jnp.tile(table, (reps, 1))
    lane = (jnp.arange(_N, dtype=jnp.int32) % reps) * _BINS
    idx = x.astype(jnp.int32).T.reshape(_N) + lane

    info = plsc.get_sparse_core_info()
    nc, ns = info.num_cores, info.num_subcores
    nw = nc * ns

    bblk = 16
    chunk = _BATCH // bblk
    n_chunks_tot = _NUM_FEATURE * bblk
    chunks_w = n_chunks_tot // nw
    out_w = _NUM_FEATURE * _EMB

    mesh = plsc.VectorSubcoreMesh(core_axis_name="c", subcore_axis_name="s")

    @functools.partial(
        pl.kernel,
        out_type=jax.ShapeDtypeStruct((_BATCH, out_w), jnp.float32),
        mesh=mesh,
        scratch_types=[
            pltpu.VMEM((2, chunk), jnp.int32),
            pltpu.VMEM((2, chunk, _EMB), jnp.float32),
            pltpu.SemaphoreType.DMA,
            pltpu.SemaphoreType.DMA,
        ],
        compiler_params=pltpu.CompilerParams(use_tc_tiling_on_sc=False),
    )
    def _gather(tab_hbm, idx_hbm, out_hbm, idx_v, rows_v, sem0, sem1):
        wid = lax.axis_index("s") * nc + lax.axis_index("c")
        g0 = wid * chunks_w

        sems = [sem0, sem1]
        handles = [None, None]
        pltpu.sync_copy(idx_hbm.at[pl.ds(g0 * chunk, chunk)], idx_v.at[0])
        handles[0] = pltpu.async_copy(
            tab_hbm.at[idx_v.at[0]], rows_v.at[0], sems[0])
        for c in range(chunks_w):
            nb = (c + 1) % 2
            if c + 1 < chunks_w:
                pltpu.sync_copy(
                    idx_hbm.at[pl.ds((g0 + c + 1) * chunk, chunk)],
                    idx_v.at[nb])
                handles[nb] = pltpu.async_copy(
                    tab_hbm.at[idx_v.at[nb]], rows_v.at[nb], sems[nb])
            g = g0 + c
            f = g // bblk
            b0 = (g % bblk) * chunk
            handles[c % 2].wait()
            pltpu.sync_copy(rows_v.at[c % 2],
                            out_hbm.at[pl.ds(b0, chunk),
                                       pl.ds(f * _EMB, _EMB)])

    return _gather(table_rep, idx)
